# baseline (speedup 1.0000x reference)
import functools

import jax
import jax.numpy as jnp
from jax import lax
from jax.experimental import pallas as pl
from jax.experimental.pallas import tpu as pltpu
from jax.experimental.pallas import tpu_sc as plsc

N_NODES = 10000
N_EDGES = 320000
DIM_IN = 128
LATENT = 64

NC = 2
NS = 16
L = 16
NW = NC * NS

K = 128
CHUNKS = -(-N_EDGES // (NW * K))
E_PAD = NW * K * CHUNKS
NPAD = 10240
STRIPE = NPAD // NS

_mesh = lambda: plsc.VectorSubcoreMesh(
    core_axis_name="c", subcore_axis_name="s", num_cores=NC, num_subcores=NS)


def _fold_body(x_ref, wg_ref, w1_ref, w3_ref, bg_ref, b1_ref, b3_ref,
               h_ref, bias_ref):
    cT = (((1,), (1,)), ((), ()))
    f32 = jnp.float32
    h = lax.dot_general(x_ref[...], wg_ref[...], cT, preferred_element_type=f32)
    h = lax.dot_general(h, w1_ref[...], cT, preferred_element_type=f32)
    h = lax.dot_general(h, w3_ref[...], cT, preferred_element_type=f32)
    h_ref[:N_NODES, :] = h
    h_ref[N_NODES:, :] = jnp.zeros((NPAD - N_NODES, LATENT), f32)
    bias = lax.dot_general(bg_ref[...], w1_ref[...], cT,
                           preferred_element_type=f32) + b1_ref[...]
    bias_ref[...] = lax.dot_general(bias, w3_ref[...], cT,
                                    preferred_element_type=f32) + b3_ref[...]


def _fold_call(x, wg, w1, w3, bg, b1, b3):
    out_shape = [
        jax.ShapeDtypeStruct((NPAD, LATENT), jnp.float32),
        jax.ShapeDtypeStruct((1, LATENT), jnp.float32),
    ]
    return pl.pallas_call(_fold_body, out_shape=out_shape)(
        x, wg, w1, w3, bg, b1, b3)


def _rsqrt_vec(d):
    i = plsc.bitcast(d, jnp.int32)
    y = plsc.bitcast(jnp.int32(0x5F3759DF) - (i >> 1), jnp.float32)
    for _ in range(3):
        y = y * (1.5 - 0.5 * d * y * y)
    return y


def _sc_body(src_hbm, dst_hbm, h_hbm, acc_out,
             src_v, dste_v, rows0_v, rows1_v, ones_v, zrow_v,
             dinv_v, hs_sh, acc_sh, deg_sh, sem0, sem1):
    cid = lax.axis_index("c")
    sid = lax.axis_index("s")
    wid = cid * NS + sid

    pltpu.sync_copy(src_hbm.at[wid], src_v)
    pltpu.sync_copy(h_hbm.at[pl.ds(sid * STRIPE, STRIPE)],
                    hs_sh.at[pl.ds(sid * STRIPE, STRIPE)])
    one = jnp.ones((L,), jnp.float32)
    zero = jnp.zeros((L,), jnp.float32)
    for i in range(K // L):
        ones_v[pl.ds(i * L, L)] = one
        zrow_v[pl.ds(i * L, L)] = zero

    def zrow(i, carry):
        for c in range(LATENT // L):
            rows0_v[i, pl.ds(c * L, L)] = zero
        return carry

    lax.fori_loop(0, K, zrow, 0)
    for r in range(STRIPE // K):
        pltpu.sync_copy(rows0_v, acc_sh.at[pl.ds(sid * STRIPE + r * K, K)])
        pltpu.sync_copy(zrow_v, deg_sh.at[pl.ds(sid * STRIPE + r * K, K)])
    plsc.subcore_barrier()

    for half in range(2):
        pltpu.sync_copy(dst_hbm.at[sid + half * NS], dste_v)
        pltpu.async_copy(ones_v, deg_sh.at[dste_v.at[0]], sem0, add=True)

        def dbody(j, carry):
            @pl.when(j + 1 < CHUNKS)
            def _():
                pltpu.async_copy(ones_v, deg_sh.at[dste_v.at[j + 1]], sem0,
                                 add=True)

            pltpu.make_async_copy(ones_v, deg_sh.at[dste_v.at[0]],
                                  sem0).wait()
            return carry

        lax.fori_loop(0, CHUNKS, dbody, 0)
    pltpu.sync_copy(dst_hbm.at[wid], dste_v)
    plsc.subcore_barrier()

    base = sid * STRIPE
    for b in range(STRIPE // K):
        pltpu.sync_copy(deg_sh.at[pl.ds(base + b * K, K)], zrow_v)
        for i in range(K // L):
            sl = pl.ds(i * L, L)
            dinv_v[sl] = _rsqrt_vec(zrow_v[sl] + 1.0)
        pltpu.sync_copy(hs_sh.at[pl.ds(base + b * K, K)], rows0_v)

        def sband(i, carry):
            dband = dinv_v[pl.ds(i * L, L)]
            for k in range(L):
                dvec = jnp.full((L,), dband[k], dtype=jnp.float32)
                r = i * L + k
                for c in range(LATENT // L):
                    sl = pl.ds(c * L, L)
                    rows0_v[r, sl] = rows0_v[r, sl] * dvec
            return carry

        lax.fori_loop(0, K // L, sband, 0)
        pltpu.sync_copy(rows0_v, hs_sh.at[pl.ds(base + b * K, K)])
    plsc.subcore_barrier()

    pltpu.async_copy(hs_sh.at[src_v.at[0]], rows0_v, sem0)

    def body(i, carry):
        j0 = 2 * i
        pltpu.async_copy(hs_sh.at[src_v.at[j0 + 1]], rows1_v, sem1)
        pltpu.make_async_copy(hs_sh.at[src_v.at[j0]], rows0_v, sem0).wait()
        pltpu.sync_copy(rows0_v, acc_sh.at[dste_v.at[j0]], add=True)

        @pl.when(j0 + 2 < CHUNKS)
        def _():
            pltpu.async_copy(hs_sh.at[src_v.at[j0 + 2]], rows0_v, sem0)

        pltpu.make_async_copy(hs_sh.at[src_v.at[j0]], rows1_v, sem1).wait()
        pltpu.sync_copy(rows1_v, acc_sh.at[dste_v.at[j0 + 1]], add=True)
        return carry

    lax.fori_loop(0, CHUNKS // 2, body, 0)

    if CHUNKS % 2 == 1:
        j = CHUNKS - 1
        pltpu.make_async_copy(hs_sh.at[src_v.at[j]], rows0_v, sem0).wait()
        pltpu.sync_copy(rows0_v, acc_sh.at[dste_v.at[j]], add=True)

    plsc.subcore_barrier()

    for b in range(STRIPE // K):
        off = base + b * K
        pltpu.sync_copy(deg_sh.at[pl.ds(off, K)], zrow_v)
        for i in range(K // L):
            sl = pl.ds(i * L, L)
            dinv_v[sl] = _rsqrt_vec(zrow_v[sl] + 1.0)
        pltpu.sync_copy(acc_sh.at[pl.ds(off, K)], rows0_v)

        @pl.when(cid == 0)
        def _():
            pltpu.sync_copy(hs_sh.at[pl.ds(off, K)], rows1_v)

            def arow(r, carry):
                for c in range(LATENT // L):
                    sl = pl.ds(c * L, L)
                    rows0_v[r, sl] = rows0_v[r, sl] + rows1_v[r, sl]
                return carry

            lax.fori_loop(0, K, arow, 0)

        def wband(i, carry):
            dband = dinv_v[pl.ds(i * L, L)]
            for k in range(L):
                dvec = jnp.full((L,), dband[k], dtype=jnp.float32)
                r = i * L + k
                for c in range(LATENT // L):
                    sl = pl.ds(c * L, L)
                    rows0_v[r, sl] = rows0_v[r, sl] * dvec
            return carry

        lax.fori_loop(0, K // L, wband, 0)
        pltpu.sync_copy(rows0_v, acc_out.at[cid, pl.ds(off, K)])


def _sc_call(src3, dst3, h):
    kern = functools.partial(
        pl.kernel,
        out_type=jax.ShapeDtypeStruct((NC, NPAD, LATENT), jnp.float32),
        mesh=_mesh(),
        scratch_types=[
            pltpu.VMEM((CHUNKS, K), jnp.int32),
            pltpu.VMEM((CHUNKS, K), jnp.int32),
            pltpu.VMEM((K, LATENT), jnp.float32),
            pltpu.VMEM((K, LATENT), jnp.float32),
            pltpu.VMEM((K,), jnp.float32),
            pltpu.VMEM((K,), jnp.float32),
            pltpu.VMEM((K,), jnp.float32),
            pltpu.VMEM_SHARED((NPAD, LATENT), jnp.float32),
            pltpu.VMEM_SHARED((NPAD, LATENT), jnp.float32),
            pltpu.VMEM_SHARED((NPAD,), jnp.float32),
            pltpu.SemaphoreType.DMA,
            pltpu.SemaphoreType.DMA,
        ],
        compiler_params=pltpu.CompilerParams(use_tc_tiling_on_sc=False,
                                             needs_layout_passes=False),
    )(_sc_body)
    return kern(src3, dst3, h)


def _combine_body(acc_ref, bias_ref, out_ref):
    out_ref[...] = acc_ref[0, :N_NODES] + acc_ref[1, :N_NODES] + bias_ref[...]


def _combine_call(acc, bias):
    return pl.pallas_call(
        _combine_body,
        out_shape=jax.ShapeDtypeStruct((N_NODES, LATENT), jnp.float32),
    )(acc, bias)


def kernel(x, edge_index, batch, W_gcn, b_gcn, W1, b1, W3, b3):
    del batch
    pad_e = E_PAD - N_EDGES
    src3 = jnp.pad(edge_index[0], (0, pad_e),
                   constant_values=N_NODES).reshape(NW, CHUNKS, K)
    dst3 = jnp.pad(edge_index[1], (0, pad_e),
                   constant_values=N_NODES).reshape(NW, CHUNKS, K)

    h, bias = _fold_call(x, W_gcn, W1, W3, b_gcn.reshape(1, DIM_IN),
                         b1.reshape(1, LATENT), b3.reshape(1, LATENT))
    acc = _sc_call(src3, dst3, h)
    return _combine_call(acc, bias)


# speedup vs baseline: 1.0246x; 1.0246x over previous
import functools

import jax
import jax.numpy as jnp
from jax import lax
from jax.experimental import pallas as pl
from jax.experimental.pallas import tpu as pltpu
from jax.experimental.pallas import tpu_sc as plsc

N_NODES = 10000
N_EDGES = 320000
DIM_IN = 128
LATENT = 64

NC = 2
NS = 16
L = 16
NW = NC * NS

K = 128
CHUNKS = -(-N_EDGES // (NW * K))
E_PAD = NW * K * CHUNKS
NPAD = 10240
STRIPE = NPAD // NS

_mesh = lambda: plsc.VectorSubcoreMesh(
    core_axis_name="c", subcore_axis_name="s", num_cores=NC, num_subcores=NS)


def _fold_body(x_ref, wg_ref, w1_ref, w3_ref, bg_ref, b1_ref, b3_ref,
               h_ref, bias_ref):
    cT = (((1,), (1,)), ((), ()))
    f32 = jnp.float32
    h = lax.dot_general(x_ref[...], wg_ref[...], cT, preferred_element_type=f32)
    h = lax.dot_general(h, w1_ref[...], cT, preferred_element_type=f32)
    h = lax.dot_general(h, w3_ref[...], cT, preferred_element_type=f32)
    h_ref[:N_NODES, :] = h
    h_ref[N_NODES:, :] = jnp.zeros((NPAD - N_NODES, LATENT), f32)
    bias = lax.dot_general(bg_ref[...], w1_ref[...], cT,
                           preferred_element_type=f32) + b1_ref[...]
    bias_ref[...] = lax.dot_general(bias, w3_ref[...], cT,
                                    preferred_element_type=f32) + b3_ref[...]


def _fold_call(x, wg, w1, w3, bg, b1, b3):
    out_shape = [
        jax.ShapeDtypeStruct((NPAD, LATENT), jnp.float32),
        jax.ShapeDtypeStruct((1, LATENT), jnp.float32),
    ]
    return pl.pallas_call(_fold_body, out_shape=out_shape)(
        x, wg, w1, w3, bg, b1, b3)


def _deg_body(dst_hbm, deg_out, idx_v, ones_v, zrow_v, deg_sh, sem):
    cid = lax.axis_index("c")
    sid = lax.axis_index("s")
    wid = cid * NS + sid
    pltpu.sync_copy(dst_hbm.at[wid], idx_v)
    one = jnp.ones((L,), jnp.float32)
    zero = jnp.zeros((L,), jnp.float32)
    for i in range(K // L):
        ones_v[pl.ds(i * L, L)] = one
        zrow_v[pl.ds(i * L, L)] = zero
    for r in range(STRIPE // K):
        pltpu.sync_copy(zrow_v, deg_sh.at[pl.ds(sid * STRIPE + r * K, K)])
    plsc.subcore_barrier()

    pltpu.async_copy(ones_v, deg_sh.at[idx_v.at[0]], sem, add=True)

    def dbody(j, carry):
        @pl.when(j + 1 < CHUNKS)
        def _():
            pltpu.async_copy(ones_v, deg_sh.at[idx_v.at[j + 1]], sem,
                             add=True)

        pltpu.make_async_copy(ones_v, deg_sh.at[idx_v.at[0]], sem).wait()
        return carry

    lax.fori_loop(0, CHUNKS, dbody, 0)
    plsc.subcore_barrier()
    pltpu.sync_copy(deg_sh.at[pl.ds(sid * STRIPE, STRIPE)],
                    deg_out.at[cid, pl.ds(sid * STRIPE, STRIPE)])


def _deg_call(dst3):
    kern = functools.partial(
        pl.kernel,
        out_type=jax.ShapeDtypeStruct((NC, NPAD), jnp.float32),
        mesh=_mesh(),
        scratch_types=[
            pltpu.VMEM((CHUNKS, K), jnp.int32),
            pltpu.VMEM((K,), jnp.float32),
            pltpu.VMEM((K,), jnp.float32),
            pltpu.VMEM_SHARED((NPAD,), jnp.float32),
            pltpu.SemaphoreType.DMA,
        ],
    )(_deg_body)
    return kern(dst3)


def _rsqrt_vec(d):
    i = plsc.bitcast(d, jnp.int32)
    y = plsc.bitcast(jnp.int32(0x5F3759DF) - (i >> 1), jnp.float32)
    for _ in range(3):
        y = y * (1.5 - 0.5 * d * y * y)
    return y


def _edge_body(src_hbm, dst_hbm, h_hbm, deg_hbm, acc_out,
               src_v, dste_v, rows0_v, rows1_v, degv, dinv_v,
               hs_sh, acc_sh, sem0, sem1):
    cid = lax.axis_index("c")
    sid = lax.axis_index("s")
    wid = cid * NS + sid
    base = sid * STRIPE
    pltpu.sync_copy(src_hbm.at[wid], src_v)
    pltpu.sync_copy(dst_hbm.at[wid], dste_v)
    zero = jnp.zeros((L,), jnp.float32)

    def zrow(i, carry):
        for c in range(LATENT // L):
            rows1_v[i, pl.ds(c * L, L)] = zero
        return carry

    lax.fori_loop(0, K, zrow, 0)

    def dinv_block(b):
        off = base + b * K
        pltpu.sync_copy(deg_hbm.at[0, pl.ds(off, K)], degv)
        pltpu.sync_copy(deg_hbm.at[1, pl.ds(off, K)], dinv_v)
        for i in range(K // L):
            sl = pl.ds(i * L, L)
            dinv_v[sl] = _rsqrt_vec(degv[sl] + dinv_v[sl] + 1.0)

    def scale_block(buf_v):
        def sband(i, carry):
            dband = dinv_v[pl.ds(i * L, L)]
            for k in range(L):
                dvec = jnp.full((L,), dband[k], dtype=jnp.float32)
                for c in range(LATENT // L):
                    sl = pl.ds(c * L, L)
                    buf_v[i * L + k, sl] = buf_v[i * L + k, sl] * dvec
            return carry

        lax.fori_loop(0, K // L, sband, 0)

    for b in range(STRIPE // K):
        off = base + b * K
        dinv_block(b)
        pltpu.sync_copy(h_hbm.at[pl.ds(off, K)], rows0_v)
        scale_block(rows0_v)
        pltpu.sync_copy(rows0_v, hs_sh.at[pl.ds(off, K)])
        pltpu.sync_copy(rows1_v, acc_sh.at[pl.ds(off, K)])
    plsc.subcore_barrier()

    pltpu.async_copy(hs_sh.at[src_v.at[0]], rows0_v, sem0)

    def body(i, carry):
        j0 = 2 * i
        pltpu.async_copy(hs_sh.at[src_v.at[j0 + 1]], rows1_v, sem1)
        pltpu.make_async_copy(hs_sh.at[src_v.at[j0]], rows0_v, sem0).wait()
        pltpu.sync_copy(rows0_v, acc_sh.at[dste_v.at[j0]], add=True)

        @pl.when(j0 + 2 < CHUNKS)
        def _():
            pltpu.async_copy(hs_sh.at[src_v.at[j0 + 2]], rows0_v, sem0)

        pltpu.make_async_copy(hs_sh.at[src_v.at[j0]], rows1_v, sem1).wait()
        pltpu.sync_copy(rows1_v, acc_sh.at[dste_v.at[j0 + 1]], add=True)
        return carry

    lax.fori_loop(0, CHUNKS // 2, body, 0)

    if CHUNKS % 2 == 1:
        j = CHUNKS - 1
        pltpu.make_async_copy(hs_sh.at[src_v.at[j]], rows0_v, sem0).wait()
        pltpu.sync_copy(rows0_v, acc_sh.at[dste_v.at[j]], add=True)
    plsc.subcore_barrier()

    for b in range(STRIPE // K):
        off = base + b * K
        dinv_block(b)
        pltpu.sync_copy(acc_sh.at[pl.ds(off, K)], rows0_v)

        @pl.when(cid == 0)
        def _():
            pltpu.sync_copy(hs_sh.at[pl.ds(off, K)], rows1_v)

            def arow(r, carry):
                for c in range(LATENT // L):
                    sl = pl.ds(c * L, L)
                    rows0_v[r, sl] = rows0_v[r, sl] + rows1_v[r, sl]
                return carry

            lax.fori_loop(0, K, arow, 0)

        scale_block(rows0_v)
        pltpu.sync_copy(rows0_v, acc_out.at[cid, pl.ds(off, K)])


def _edge_call(src3, dst3, h, deg):
    kern = functools.partial(
        pl.kernel,
        out_type=jax.ShapeDtypeStruct((NC, NPAD, LATENT), jnp.float32),
        mesh=_mesh(),
        scratch_types=[
            pltpu.VMEM((CHUNKS, K), jnp.int32),
            pltpu.VMEM((CHUNKS, K), jnp.int32),
            pltpu.VMEM((K, LATENT), jnp.float32),
            pltpu.VMEM((K, LATENT), jnp.float32),
            pltpu.VMEM((K,), jnp.float32),
            pltpu.VMEM((K,), jnp.float32),
            pltpu.VMEM_SHARED((NPAD, LATENT), jnp.float32),
            pltpu.VMEM_SHARED((NPAD, LATENT), jnp.float32),
            pltpu.SemaphoreType.DMA,
            pltpu.SemaphoreType.DMA,
        ],
        compiler_params=pltpu.CompilerParams(use_tc_tiling_on_sc=False,
                                             needs_layout_passes=False),
    )(_edge_body)
    return kern(src3, dst3, h, deg)


def _combine_body(acc_ref, bias_ref, out_ref):
    out_ref[...] = acc_ref[0, :N_NODES] + acc_ref[1, :N_NODES] + bias_ref[...]


def _combine_call(acc, bias):
    return pl.pallas_call(
        _combine_body,
        out_shape=jax.ShapeDtypeStruct((N_NODES, LATENT), jnp.float32),
    )(acc, bias)


def kernel(x, edge_index, batch, W_gcn, b_gcn, W1, b1, W3, b3):
    del batch
    pad_e = E_PAD - N_EDGES
    src3 = jnp.pad(edge_index[0], (0, pad_e),
                   constant_values=N_NODES).reshape(NW, CHUNKS, K)
    dst3 = jnp.pad(edge_index[1], (0, pad_e),
                   constant_values=N_NODES).reshape(NW, CHUNKS, K)

    h, bias = _fold_call(x, W_gcn, W1, W3, b_gcn.reshape(1, DIM_IN),
                         b1.reshape(1, LATENT), b3.reshape(1, LATENT))
    deg = _deg_call(dst3)
    acc = _edge_call(src3, dst3, h, deg)
    return _combine_call(acc, bias)


# speedup vs baseline: 1.1570x; 1.1291x over previous
import functools

import jax
import jax.numpy as jnp
from jax import lax
from jax.experimental import pallas as pl
from jax.experimental.pallas import tpu as pltpu
from jax.experimental.pallas import tpu_sc as plsc

N_NODES = 10000
N_EDGES = 320000
DIM_IN = 128
LATENT = 64

NC = 2
NS = 16
L = 16
NW = NC * NS

K = 128
CHUNKS = -(-N_EDGES // (NW * K))
E_PAD = NW * K * CHUNKS
NPAD = 10240
STRIPE = NPAD // NS

_mesh = lambda: plsc.VectorSubcoreMesh(
    core_axis_name="c", subcore_axis_name="s", num_cores=NC, num_subcores=NS)


def _deg_body(dst_hbm, deg_out, idx_v, ones_v, zrow_v, deg_sh, sem):
    cid = lax.axis_index("c")
    sid = lax.axis_index("s")
    wid = cid * NS + sid
    pltpu.sync_copy(dst_hbm.at[wid], idx_v)
    one = jnp.ones((L,), jnp.float32)
    zero = jnp.zeros((L,), jnp.float32)
    for i in range(K // L):
        ones_v[pl.ds(i * L, L)] = one
        zrow_v[pl.ds(i * L, L)] = zero
    for r in range(STRIPE // K):
        pltpu.sync_copy(zrow_v, deg_sh.at[pl.ds(sid * STRIPE + r * K, K)])
    plsc.subcore_barrier()

    pltpu.async_copy(ones_v, deg_sh.at[idx_v.at[0]], sem, add=True)

    def dbody(j, carry):
        @pl.when(j + 1 < CHUNKS)
        def _():
            pltpu.async_copy(ones_v, deg_sh.at[idx_v.at[j + 1]], sem,
                             add=True)

        pltpu.make_async_copy(ones_v, deg_sh.at[idx_v.at[0]], sem).wait()
        return carry

    lax.fori_loop(0, CHUNKS, dbody, 0)
    plsc.subcore_barrier()
    pltpu.sync_copy(deg_sh.at[pl.ds(sid * STRIPE, STRIPE)],
                    deg_out.at[cid, pl.ds(sid * STRIPE, STRIPE)])


def _deg_call(dst3):
    kern = functools.partial(
        pl.kernel,
        out_type=jax.ShapeDtypeStruct((NC, NPAD), jnp.float32),
        mesh=_mesh(),
        scratch_types=[
            pltpu.VMEM((CHUNKS, K), jnp.int32),
            pltpu.VMEM((K,), jnp.float32),
            pltpu.VMEM((K,), jnp.float32),
            pltpu.VMEM_SHARED((NPAD,), jnp.float32),
            pltpu.SemaphoreType.DMA,
        ],
    )(_deg_body)
    return kern(dst3)


def _fold_body(x_ref, wg_ref, w1_ref, w3_ref, bg_ref, b1_ref, b3_ref,
               deg_ref, hs_ref, selfb_ref):
    cT = (((1,), (1,)), ((), ()))
    f32 = jnp.float32
    h = lax.dot_general(x_ref[...], wg_ref[...], cT, preferred_element_type=f32)
    h = lax.dot_general(h, w1_ref[...], cT, preferred_element_type=f32)
    h = lax.dot_general(h, w3_ref[...], cT, preferred_element_type=f32)
    bias = lax.dot_general(bg_ref[...], w1_ref[...], cT,
                           preferred_element_type=f32) + b1_ref[...]
    bias = lax.dot_general(bias, w3_ref[...], cT,
                           preferred_element_type=f32) + b3_ref[...]
    deg = deg_ref[:N_NODES, 0:1] + deg_ref[:N_NODES, 1:2] + 1.0
    dinv = lax.rsqrt(deg)
    hs_ref[:N_NODES, :] = h * dinv
    hs_ref[N_NODES:, :] = jnp.zeros((NPAD - N_NODES, LATENT), f32)
    selfb_ref[:N_NODES, :] = h * (dinv * dinv) + bias
    selfb_ref[N_NODES:, :] = jnp.zeros((NPAD - N_NODES, LATENT), f32)


def _fold_call(x, wg, w1, w3, bg, b1, b3, deg_t):
    out_shape = [
        jax.ShapeDtypeStruct((NPAD, LATENT), jnp.float32),
        jax.ShapeDtypeStruct((NPAD, LATENT), jnp.float32),
    ]
    return pl.pallas_call(_fold_body, out_shape=out_shape)(
        x, wg, w1, w3, bg, b1, b3, deg_t)


def _edge_body(src_hbm, dst_hbm, hs_hbm, acc_out,
               src_v, dste_v, rows0_v, rows1_v, hs_sh, acc_sh, sem0, sem1):
    cid = lax.axis_index("c")
    sid = lax.axis_index("s")
    wid = cid * NS + sid
    base = sid * STRIPE
    pltpu.sync_copy(src_hbm.at[wid], src_v)
    pltpu.sync_copy(dst_hbm.at[wid], dste_v)
    pltpu.sync_copy(hs_hbm.at[pl.ds(base, STRIPE)],
                    hs_sh.at[pl.ds(base, STRIPE)])
    zero = jnp.zeros((L,), jnp.float32)

    def zrow(i, carry):
        for c in range(LATENT // L):
            rows0_v[i, pl.ds(c * L, L)] = zero
        return carry

    lax.fori_loop(0, K, zrow, 0)
    for r in range(STRIPE // K):
        pltpu.sync_copy(rows0_v, acc_sh.at[pl.ds(base + r * K, K)])
    plsc.subcore_barrier()

    pltpu.async_copy(hs_sh.at[src_v.at[0]], rows0_v, sem0)

    def body(i, carry):
        j0 = 2 * i
        pltpu.async_copy(hs_sh.at[src_v.at[j0 + 1]], rows1_v, sem1)
        pltpu.make_async_copy(hs_sh.at[src_v.at[j0]], rows0_v, sem0).wait()
        pltpu.sync_copy(rows0_v, acc_sh.at[dste_v.at[j0]], add=True)

        @pl.when(j0 + 2 < CHUNKS)
        def _():
            pltpu.async_copy(hs_sh.at[src_v.at[j0 + 2]], rows0_v, sem0)

        pltpu.make_async_copy(hs_sh.at[src_v.at[j0]], rows1_v, sem1).wait()
        pltpu.sync_copy(rows1_v, acc_sh.at[dste_v.at[j0 + 1]], add=True)
        return carry

    lax.fori_loop(0, CHUNKS // 2, body, 0)

    if CHUNKS % 2 == 1:
        j = CHUNKS - 1
        pltpu.make_async_copy(hs_sh.at[src_v.at[j]], rows0_v, sem0).wait()
        pltpu.sync_copy(rows0_v, acc_sh.at[dste_v.at[j]], add=True)

    plsc.subcore_barrier()
    pltpu.sync_copy(acc_sh.at[pl.ds(base, STRIPE)],
                    acc_out.at[cid, pl.ds(base, STRIPE)])


def _edge_call(src3, dst3, hs):
    kern = functools.partial(
        pl.kernel,
        out_type=jax.ShapeDtypeStruct((NC, NPAD, LATENT), jnp.float32),
        mesh=_mesh(),
        scratch_types=[
            pltpu.VMEM((CHUNKS, K), jnp.int32),
            pltpu.VMEM((CHUNKS, K), jnp.int32),
            pltpu.VMEM((K, LATENT), jnp.float32),
            pltpu.VMEM((K, LATENT), jnp.float32),
            pltpu.VMEM_SHARED((NPAD, LATENT), jnp.float32),
            pltpu.VMEM_SHARED((NPAD, LATENT), jnp.float32),
            pltpu.SemaphoreType.DMA,
            pltpu.SemaphoreType.DMA,
        ],
        compiler_params=pltpu.CompilerParams(use_tc_tiling_on_sc=False),
    )(_edge_body)
    return kern(src3, dst3, hs)


def _combine_body(acc_ref, deg_ref, selfb_ref, out_ref):
    deg = deg_ref[:N_NODES, 0:1] + deg_ref[:N_NODES, 1:2] + 1.0
    dinv = lax.rsqrt(deg)
    a = acc_ref[0, :N_NODES] + acc_ref[1, :N_NODES]
    out_ref[...] = a * dinv + selfb_ref[:N_NODES]


def _combine_call(acc, deg_t, selfb):
    return pl.pallas_call(
        _combine_body,
        out_shape=jax.ShapeDtypeStruct((N_NODES, LATENT), jnp.float32),
    )(acc, deg_t, selfb)


def kernel(x, edge_index, batch, W_gcn, b_gcn, W1, b1, W3, b3):
    del batch
    pad_e = E_PAD - N_EDGES
    src3 = jnp.pad(edge_index[0], (0, pad_e),
                   constant_values=N_NODES).reshape(NW, CHUNKS, K)
    dst3 = jnp.pad(edge_index[1], (0, pad_e),
                   constant_values=N_NODES).reshape(NW, CHUNKS, K)

    deg = _deg_call(dst3)
    deg_t = deg.T
    hs, selfb = _fold_call(x, W_gcn, W1, W3, b_gcn.reshape(1, DIM_IN),
                           b1.reshape(1, LATENT), b3.reshape(1, LATENT),
                           deg_t)
    acc = _edge_call(src3, dst3, hs)
    return _combine_call(acc, deg_t, selfb)
